# initial kernel scaffold (unmeasured)
import jax
import jax.numpy as jnp
from jax import lax
from jax.experimental import pallas as pl
from jax.experimental.pallas import tpu as pltpu


def kernel(
    x,
):
    def body(*refs):
        pass

    out_shape = jax.ShapeDtypeStruct(..., jnp.float32)
    return pl.pallas_call(body, out_shape=out_shape)(...)


# baseline (device time: 19012 ns/iter reference)
import jax
import jax.numpy as jnp
from jax import lax
from jax.experimental import pallas as pl
from jax.experimental.pallas import tpu as pltpu

N_Y = 4


def kernel(x):
    m, n = x.shape

    def body(x_ref, out_ref, buf, send_sems, recv_sems):
        my_x = lax.axis_index("x")
        my_y = lax.axis_index("y")
        my_z = lax.axis_index("z")
        left = (my_y - 1) % N_Y
        right = (my_y + 1) % N_Y

        barrier = pltpu.get_barrier_semaphore()
        for nbr in (left, right):
            pl.semaphore_signal(
                barrier, inc=1,
                device_id=(my_x, nbr, my_z),
                device_id_type=pl.DeviceIdType.MESH,
            )
        pl.semaphore_wait(barrier, 2)

        buf[0] = x_ref[...]
        acc = x_ref[...]
        for h in range(N_Y - 1):
            rdma = pltpu.make_async_remote_copy(
                src_ref=buf.at[h],
                dst_ref=buf.at[h + 1],
                send_sem=send_sems.at[h],
                recv_sem=recv_sems.at[h],
                device_id=(my_x, right, my_z),
                device_id_type=pl.DeviceIdType.MESH,
            )
            rdma.start()
            rdma.wait()
            acc = acc + buf[h + 1]
        out_ref[...] = acc

    return pl.pallas_call(
        body,
        out_shape=jax.ShapeDtypeStruct((m, n), x.dtype),
        in_specs=[pl.BlockSpec(memory_space=pltpu.VMEM)],
        out_specs=pl.BlockSpec(memory_space=pltpu.VMEM),
        scratch_shapes=[
            pltpu.VMEM((N_Y, m, n), x.dtype),
            pltpu.SemaphoreType.DMA((N_Y - 1,)),
            pltpu.SemaphoreType.DMA((N_Y - 1,)),
        ],
        compiler_params=pltpu.CompilerParams(collective_id=0),
    )(x)


# device time: 16906 ns/iter; 1.1246x vs baseline; 1.1246x over previous
import jax
import jax.numpy as jnp
from jax import lax
from jax.experimental import pallas as pl
from jax.experimental.pallas import tpu as pltpu

N_Y = 4


def kernel(x):
    m, n = x.shape

    def body(x_ref, out_ref, rbuf, sbuf, send_sems, recv_sems):
        my_x = lax.axis_index("x")
        my_y = lax.axis_index("y")
        my_z = lax.axis_index("z")
        p1 = my_y ^ 1
        p2 = my_y ^ 2

        barrier = pltpu.get_barrier_semaphore()
        for p in (p1, p2):
            pl.semaphore_signal(
                barrier, inc=1,
                device_id=(my_x, p, my_z),
                device_id_type=pl.DeviceIdType.MESH,
            )
        pl.semaphore_wait(barrier, 2)

        r1 = pltpu.make_async_remote_copy(
            src_ref=x_ref,
            dst_ref=rbuf.at[0],
            send_sem=send_sems.at[0],
            recv_sem=recv_sems.at[0],
            device_id=(my_x, p1, my_z),
            device_id_type=pl.DeviceIdType.MESH,
        )
        r1.start()
        r1.wait()
        sbuf[...] = x_ref[...] + rbuf[0]

        r2 = pltpu.make_async_remote_copy(
            src_ref=sbuf,
            dst_ref=rbuf.at[1],
            send_sem=send_sems.at[1],
            recv_sem=recv_sems.at[1],
            device_id=(my_x, p2, my_z),
            device_id_type=pl.DeviceIdType.MESH,
        )
        r2.start()
        r2.wait()
        out_ref[...] = sbuf[...] + rbuf[1]

    return pl.pallas_call(
        body,
        out_shape=jax.ShapeDtypeStruct((m, n), x.dtype),
        in_specs=[pl.BlockSpec(memory_space=pltpu.VMEM)],
        out_specs=pl.BlockSpec(memory_space=pltpu.VMEM),
        scratch_shapes=[
            pltpu.VMEM((2, m, n), x.dtype),
            pltpu.VMEM((m, n), x.dtype),
            pltpu.SemaphoreType.DMA((2,)),
            pltpu.SemaphoreType.DMA((2,)),
        ],
        compiler_params=pltpu.CompilerParams(collective_id=0),
    )(x)


# device time: 15485 ns/iter; 1.2278x vs baseline; 1.0918x over previous
import jax
import jax.numpy as jnp
from jax import lax
from jax.experimental import pallas as pl
from jax.experimental.pallas import tpu as pltpu

N_Y = 4
N_CHUNK = 2


def kernel(x):
    m, n = x.shape
    m2 = m // N_CHUNK

    def body(x_ref, out_ref, rbuf, sbuf, send_sems, recv_sems):
        my_x = lax.axis_index("x")
        my_y = lax.axis_index("y")
        my_z = lax.axis_index("z")
        p1 = my_y ^ 1
        p2 = my_y ^ 2

        barrier = pltpu.get_barrier_semaphore()
        for p in (p1, p2):
            pl.semaphore_signal(
                barrier, inc=1,
                device_id=(my_x, p, my_z),
                device_id_type=pl.DeviceIdType.MESH,
            )
        pl.semaphore_wait(barrier, 2)

        r1 = []
        for c in range(N_CHUNK):
            r = pltpu.make_async_remote_copy(
                src_ref=x_ref.at[pl.ds(c * m2, m2)],
                dst_ref=rbuf.at[c],
                send_sem=send_sems.at[c],
                recv_sem=recv_sems.at[c],
                device_id=(my_x, p1, my_z),
                device_id_type=pl.DeviceIdType.MESH,
            )
            r.start()
            r1.append(r)

        r2 = []
        for c in range(N_CHUNK):
            r1[c].wait()
            sbuf[c] = x_ref[pl.ds(c * m2, m2), :] + rbuf[c]
            r = pltpu.make_async_remote_copy(
                src_ref=sbuf.at[c],
                dst_ref=rbuf.at[N_CHUNK + c],
                send_sem=send_sems.at[N_CHUNK + c],
                recv_sem=recv_sems.at[N_CHUNK + c],
                device_id=(my_x, p2, my_z),
                device_id_type=pl.DeviceIdType.MESH,
            )
            r.start()
            r2.append(r)

        for c in range(N_CHUNK):
            r2[c].wait()
            out_ref[pl.ds(c * m2, m2), :] = sbuf[c] + rbuf[N_CHUNK + c]

    return pl.pallas_call(
        body,
        out_shape=jax.ShapeDtypeStruct((m, n), x.dtype),
        in_specs=[pl.BlockSpec(memory_space=pltpu.VMEM)],
        out_specs=pl.BlockSpec(memory_space=pltpu.VMEM),
        scratch_shapes=[
            pltpu.VMEM((2 * N_CHUNK, m2, n), x.dtype),
            pltpu.VMEM((N_CHUNK, m2, n), x.dtype),
            pltpu.SemaphoreType.DMA((2 * N_CHUNK,)),
            pltpu.SemaphoreType.DMA((2 * N_CHUNK,)),
        ],
        compiler_params=pltpu.CompilerParams(collective_id=0),
    )(x)


# device time: 14644 ns/iter; 1.2983x vs baseline; 1.0574x over previous
import jax
import jax.numpy as jnp
from jax import lax
from jax.experimental import pallas as pl
from jax.experimental.pallas import tpu as pltpu

N_Y = 4
N_CHUNK = 4


def kernel(x):
    m, n = x.shape
    mh = m // 2
    mc = mh // N_CHUNK

    def body(x_ref, out_ref, rbuf1, rbuf2, sbuf, send_sems, recv_sems):
        my_x = lax.axis_index("x")
        my_y = lax.axis_index("y")
        my_z = lax.axis_index("z")
        p1 = my_y ^ 1
        p2 = my_y ^ 2
        px = 1 - my_x
        base = my_x * mh

        barrier = pltpu.get_barrier_semaphore()
        for dev in ((my_x, p1, my_z), (my_x, p2, my_z), (px, my_y, my_z)):
            pl.semaphore_signal(
                barrier, inc=1,
                device_id=dev,
                device_id_type=pl.DeviceIdType.MESH,
            )
        pl.semaphore_wait(barrier, 3)

        r1 = []
        for c in range(N_CHUNK):
            r = pltpu.make_async_remote_copy(
                src_ref=x_ref.at[pl.ds(base + c * mc, mc)],
                dst_ref=rbuf1.at[c],
                send_sem=send_sems.at[c],
                recv_sem=recv_sems.at[c],
                device_id=(my_x, p1, my_z),
                device_id_type=pl.DeviceIdType.MESH,
            )
            r.start()
            r1.append(r)

        r2 = []
        for c in range(N_CHUNK):
            r1[c].wait()
            sbuf[c] = x_ref[pl.ds(base + c * mc, mc), :] + rbuf1[c]
            r = pltpu.make_async_remote_copy(
                src_ref=sbuf.at[c],
                dst_ref=rbuf2.at[c],
                send_sem=send_sems.at[N_CHUNK + c],
                recv_sem=recv_sems.at[N_CHUNK + c],
                device_id=(my_x, p2, my_z),
                device_id_type=pl.DeviceIdType.MESH,
            )
            r.start()
            r2.append(r)

        r3 = []
        for c in range(N_CHUNK):
            r2[c].wait()
            out_ref[pl.ds(base + c * mc, mc), :] = sbuf[c] + rbuf2[c]
            r = pltpu.make_async_remote_copy(
                src_ref=out_ref.at[pl.ds(base + c * mc, mc)],
                dst_ref=out_ref.at[pl.ds(base + c * mc, mc)],
                send_sem=send_sems.at[2 * N_CHUNK + c],
                recv_sem=recv_sems.at[2 * N_CHUNK + c],
                device_id=(px, my_y, my_z),
                device_id_type=pl.DeviceIdType.MESH,
            )
            r.start()
            r3.append(r)

        for c in range(N_CHUNK):
            r3[c].wait()

    return pl.pallas_call(
        body,
        out_shape=jax.ShapeDtypeStruct((m, n), x.dtype),
        in_specs=[pl.BlockSpec(memory_space=pltpu.VMEM)],
        out_specs=pl.BlockSpec(memory_space=pltpu.VMEM),
        scratch_shapes=[
            pltpu.VMEM((N_CHUNK, mc, n), x.dtype),
            pltpu.VMEM((N_CHUNK, mc, n), x.dtype),
            pltpu.VMEM((N_CHUNK, mc, n), x.dtype),
            pltpu.SemaphoreType.DMA((3 * N_CHUNK,)),
            pltpu.SemaphoreType.DMA((3 * N_CHUNK,)),
        ],
        compiler_params=pltpu.CompilerParams(collective_id=0),
    )(x)


# device time: 14525 ns/iter; 1.3089x vs baseline; 1.0082x over previous
import jax
import jax.numpy as jnp
from jax import lax
from jax.experimental import pallas as pl
from jax.experimental.pallas import tpu as pltpu

N_Y = 4
N_CHUNK = 8


def kernel(x):
    m, n = x.shape
    mh = m // 2
    mc = mh // N_CHUNK

    def body(x_ref, out_ref, rbuf1, rbuf2, sbuf, send_sems, recv_sems):
        my_x = lax.axis_index("x")
        my_y = lax.axis_index("y")
        my_z = lax.axis_index("z")
        p1 = my_y ^ 1
        p2 = my_y ^ 2
        px = 1 - my_x
        base = my_x * mh

        barrier = pltpu.get_barrier_semaphore()
        for dev in ((my_x, p1, my_z), (my_x, p2, my_z), (px, my_y, my_z)):
            pl.semaphore_signal(
                barrier, inc=1,
                device_id=dev,
                device_id_type=pl.DeviceIdType.MESH,
            )
        pl.semaphore_wait(barrier, 3)

        r1 = []
        for c in range(N_CHUNK):
            r = pltpu.make_async_remote_copy(
                src_ref=x_ref.at[pl.ds(base + c * mc, mc)],
                dst_ref=rbuf1.at[c],
                send_sem=send_sems.at[c],
                recv_sem=recv_sems.at[c],
                device_id=(my_x, p1, my_z),
                device_id_type=pl.DeviceIdType.MESH,
            )
            r.start()
            r1.append(r)

        r2 = []
        for c in range(N_CHUNK):
            r1[c].wait()
            sbuf[c] = x_ref[pl.ds(base + c * mc, mc), :] + rbuf1[c]
            r = pltpu.make_async_remote_copy(
                src_ref=sbuf.at[c],
                dst_ref=rbuf2.at[c],
                send_sem=send_sems.at[N_CHUNK + c],
                recv_sem=recv_sems.at[N_CHUNK + c],
                device_id=(my_x, p2, my_z),
                device_id_type=pl.DeviceIdType.MESH,
            )
            r.start()
            r2.append(r)

        r3 = []
        for c in range(N_CHUNK):
            r2[c].wait()
            out_ref[pl.ds(base + c * mc, mc), :] = sbuf[c] + rbuf2[c]
            r = pltpu.make_async_remote_copy(
                src_ref=out_ref.at[pl.ds(base + c * mc, mc)],
                dst_ref=out_ref.at[pl.ds(base + c * mc, mc)],
                send_sem=send_sems.at[2 * N_CHUNK + c],
                recv_sem=recv_sems.at[2 * N_CHUNK + c],
                device_id=(px, my_y, my_z),
                device_id_type=pl.DeviceIdType.MESH,
            )
            r.start()
            r3.append(r)

        for c in range(N_CHUNK):
            r3[c].wait()

    return pl.pallas_call(
        body,
        out_shape=jax.ShapeDtypeStruct((m, n), x.dtype),
        in_specs=[pl.BlockSpec(memory_space=pltpu.VMEM)],
        out_specs=pl.BlockSpec(memory_space=pltpu.VMEM),
        scratch_shapes=[
            pltpu.VMEM((N_CHUNK, mc, n), x.dtype),
            pltpu.VMEM((N_CHUNK, mc, n), x.dtype),
            pltpu.VMEM((N_CHUNK, mc, n), x.dtype),
            pltpu.SemaphoreType.DMA((3 * N_CHUNK,)),
            pltpu.SemaphoreType.DMA((3 * N_CHUNK,)),
        ],
        compiler_params=pltpu.CompilerParams(collective_id=0),
    )(x)
